# Initial kernel scaffold; baseline (speedup 1.0000x reference)
import jax
import jax.numpy as jnp
from jax.experimental import pallas as pl


def kernel(x, z, tables, W1, b1, W2, b2, W3, b3):
    raise NotImplementedError("write your pallas kernel here")


# speedup vs baseline: 7.7519x; 7.7519x over previous
import functools

import numpy as np
import jax
import jax.numpy as jnp
from jax import lax
from jax.experimental import pallas as pl
from jax.experimental.pallas import tpu as pltpu
from jax.experimental.pallas import tpu_sc as plsc

D = 11
DD = D + 1
NLEV = 16
NFEAT = 2
HS = 2 ** 19
NPTS = 131072

_PRIMES_U32 = np.array([1, 2654435761, 805459861, 3674653429, 2097192037,
                        1434869437, 2165219737, 2654435741, 2246822519,
                        3266489917, 668265263], dtype=np.uint64)[:D]
_PRIMES_I32 = [int(np.uint32(p).view(np.int32)) for p in np.uint32(_PRIMES_U32)]
_SF = [float(np.float32(DD / np.sqrt((i + 1.0) * (i + 2.0)))) for i in range(D)]
_SCALES = np.array([16.0 * (2048.0 / 16.0) ** (l / (NLEV - 1.0))
                    for l in range(NLEV)], dtype=np.float32)

PB = 64
SB = 16
LN = 128


def _wrap_i32(v: int) -> int:
    return int(np.uint32(v % (1 << 32)).view(np.int32))


def _enc_body(scale_ref, pos_ref, idx_ref, bary_ref):
    lvl = pl.program_id(0)
    scale = scale_ref[lvl, 0]
    pos = [pos_ref[j, 0] for j in range(D)]

    c = [(pos[j] * scale) * _SF[j] for j in range(D)]
    rev = [None] * D
    rev[D - 1] = c[D - 1]
    for j in range(D - 2, -1, -1):
        rev[j] = rev[j + 1] + c[j]
    elev = [rev[0]]
    for k in range(1, D):
        elev.append(rev[k] - float(k) * c[k - 1])
    zero = jnp.zeros_like(pos[0])
    elev.append(zero - float(D) * c[D - 1])

    v = [e / 12.0 for e in elev]
    greedy = []
    for k in range(DD):
        up = jnp.ceil(v[k]) * 12.0
        down = jnp.floor(v[k]) * 12.0
        greedy.append(jnp.where(up - elev[k] < elev[k] - down, up, down))
    ssum = greedy[0]
    for k in range(1, DD):
        ssum = ssum + greedy[k]
    cs = jnp.round(ssum / 12.0).astype(jnp.int32)

    diff = [elev[k] - greedy[k] for k in range(DD)]
    rank = []
    for i in range(DD):
        acc = cs
        for j in range(DD):
            if j == i:
                continue
            cond = (diff[j] >= diff[i]) if j < i else (diff[j] > diff[i])
            acc = acc + cond.astype(jnp.int32)
        rank.append(acc)
    for k in range(DD):
        tl = rank[k] < 0
        th = rank[k] >= DD
        greedy[k] = jnp.where(tl, greedy[k] + 12.0,
                              jnp.where(th, greedy[k] - 12.0, greedy[k]))
        rank[k] = jnp.where(tl, rank[k] + DD,
                            jnp.where(th, rank[k] - DD, rank[k]))

    t = [(elev[k] - greedy[k]) / 12.0 for k in range(DD)]

    A = []
    for m in range(DD):
        am = zero
        for k in range(DD):
            am = am + jnp.where(rank[k] == m, t[k], 0.0)
        A.append(am)
    bary = [None] * DD
    bary[0] = A[D] + (1.0 + (zero - A[0]))
    for r in range(1, DD):
        bary[r] = A[D - r] - A[DD - r]

    gi = [greedy[k].astype(jnp.int32) for k in range(DD)]
    gp = [gi[k] * np.int32(_PRIMES_I32[k]) for k in range(D)]
    lvl_off = lax.shift_left(lvl, 19)
    for r in range(DD):
        h = None
        for k in range(D):
            c_hi = np.int32(_wrap_i32(r * _PRIMES_I32[k]))
            c_lo = np.int32(_wrap_i32((r - DD) * _PRIMES_I32[k]))
            term = gp[k] + jnp.where(rank[k] > (D - r), c_lo, c_hi)
            h = term if h is None else h ^ term
        fidx = (h & np.int32(HS - 1)) + lvl_off
        idx_ref[0, 0, r] = fidx
        bary_ref[0, 0, r] = bary[r]


def _encode(posR, scales):
    grid = (NLEV, PB)
    out_shape = [
        jax.ShapeDtypeStruct((NLEV, PB, DD, SB, LN), jnp.int32),
        jax.ShapeDtypeStruct((NLEV, PB, DD, SB, LN), jnp.float32),
    ]
    return pl.pallas_call(
        _enc_body,
        grid=grid,
        in_specs=[
            pl.BlockSpec((NLEV, 1), lambda l, b: (0, 0),
                         memory_space=pltpu.SMEM),
            pl.BlockSpec((D, 1, SB, LN), lambda l, b: (0, b, 0, 0)),
        ],
        out_specs=[
            pl.BlockSpec((1, 1, DD, SB, LN), lambda l, b: (l, b, 0, 0, 0)),
            pl.BlockSpec((1, 1, DD, SB, LN), lambda l, b: (l, b, 0, 0, 0)),
        ],
        out_shape=out_shape,
    )(scales, posR)


def kernel(x, z, tables, W1, b1, W2, b2, W3, b3):
    pos = jnp.concatenate([x / 2.0 + 0.5, z], axis=-1)
    posR = pos.T.reshape(D, PB, SB, LN)
    scales = jnp.asarray(_SCALES).reshape(NLEV, 1)
    idx, bary = _encode(posR, scales)

    tab = tables.reshape(NLEV * HS, NFEAT)
    rows = tab[idx]
    feats = (bary[..., None] * rows).sum(axis=2)
    h = feats.transpose(1, 2, 3, 0, 4).reshape(NPTS, NLEV * NFEAT)
    h1 = jax.nn.relu(h @ W1 + b1)
    h2 = jax.nn.relu(h1 @ W2 + b2)
    return (h2 @ W3 + b3).squeeze(-1)
